# baseline (speedup 1.0000x reference)
import jax
import jax.numpy as jnp
from jax import lax
from jax.experimental import pallas as pl
from jax.experimental.pallas import tpu as pltpu
from jax.experimental.pallas import tpu_sc as plsc

_N = 10000
_E = 320000
_D = 128
_DE = 4
_NB = 3
_NC = 2
_NS = 16
_NW = _NC * _NS
_EPW = _E // _NW
_CPT = 80
_NCH = _EPW // _CPT
_NG = _CPT // 16
_NPAD = 10240
_RPS = _NPAD // _NS
_ZR = 32

_BLK = 400
_GRID = _N // _BLK


def _matTb(h, W, b):
    return lax.dot_general(h, W, (((1,), (1,)), ((), ())),
                           preferred_element_type=jnp.float32) + b


def _layernorm(h, g, b):
    mu = jnp.mean(h, axis=-1, keepdims=True)
    var = jnp.mean((h - mu) ** 2, axis=-1, keepdims=True)
    return (h - mu) / jnp.sqrt(var + 1e-5) * g + b


def _enc_body(x_ref, W_ref, b_ref, g_ref, bb_ref, o_ref):
    h = x_ref[...]
    for i in range(3):
        h = _matTb(h, W_ref[i], b_ref[i])
        if i < 2:
            h = jnp.where(h > 0, h, 0.01 * h)
    o_ref[...] = _layernorm(h, g_ref[...], bb_ref[...])


def _enc_call(x, W, b, g, bb):
    return pl.pallas_call(
        _enc_body,
        grid=(_GRID,),
        in_specs=[
            pl.BlockSpec((_BLK, _D), lambda i: (i, 0)),
            pl.BlockSpec((3, _D, _D), lambda i: (0, 0, 0)),
            pl.BlockSpec((3, _D), lambda i: (0, 0)),
            pl.BlockSpec((1, _D), lambda i: (0, 0)),
            pl.BlockSpec((1, _D), lambda i: (0, 0)),
        ],
        out_specs=pl.BlockSpec((_BLK, _D), lambda i: (i, 0)),
        out_shape=jax.ShapeDtypeStruct((_N, _D), jnp.float32),
    )(x, W, b, g, bb)


def _prep_body(y_ref, Wl_ref, bl_ref, Wr_ref, br_ref, xl_ref, xr_ref):
    y = y_ref[...]
    xl_ref[...] = _matTb(y, Wl_ref[...], bl_ref[...])
    xr_ref[...] = _matTb(y, Wr_ref[...], br_ref[...])


def _prep_call(y, Wl, bl, Wr, br):
    return pl.pallas_call(
        _prep_body,
        grid=(_GRID,),
        in_specs=[
            pl.BlockSpec((_BLK, _D), lambda i: (i, 0)),
            pl.BlockSpec((_D, _D), lambda i: (0, 0)),
            pl.BlockSpec((1, _D), lambda i: (0, 0)),
            pl.BlockSpec((_D, _D), lambda i: (0, 0)),
            pl.BlockSpec((1, _D), lambda i: (0, 0)),
        ],
        out_specs=[
            pl.BlockSpec((_BLK, _D), lambda i: (i, 0)),
            pl.BlockSpec((_BLK, _D), lambda i: (i, 0)),
        ],
        out_shape=[
            jax.ShapeDtypeStruct((_N, _D), jnp.float32),
            jax.ShapeDtypeStruct((_N, _D), jnp.float32),
        ],
    )(y, Wl, bl, Wr, br)


def _post_body(y_ref, p0_ref, p1_ref, den_ref, bias_ref, g_ref, b_ref,
               o_ref):
    num = p0_ref[...] + p1_ref[...]
    den = jnp.sum(den_ref[...], axis=1, keepdims=True)
    gat = num / (den + 1e-16) + bias_ref[...]
    o_ref[...] = y_ref[...] + _layernorm(gat, g_ref[...], b_ref[...])


def _post_call(y, p0, p1, den_t, bias, g, b):
    return pl.pallas_call(
        _post_body,
        grid=(_GRID,),
        in_specs=[
            pl.BlockSpec((_BLK, _D), lambda i: (i, 0)),
            pl.BlockSpec((_BLK, _D), lambda i: (i, 0)),
            pl.BlockSpec((_BLK, _D), lambda i: (i, 0)),
            pl.BlockSpec((_BLK, _NW), lambda i: (i, 0)),
            pl.BlockSpec((1, _D), lambda i: (0, 0)),
            pl.BlockSpec((1, _D), lambda i: (0, 0)),
            pl.BlockSpec((1, _D), lambda i: (0, 0)),
        ],
        out_specs=pl.BlockSpec((_BLK, _D), lambda i: (i, 0)),
        out_shape=jax.ShapeDtypeStruct((_N, _D), jnp.float32),
    )(y, p0, p1, den_t, bias, g, b)


def _dec_body(x_ref, W_ref, b_ref, o_ref):
    h = x_ref[...]
    for i in range(3):
        h = _matTb(h, W_ref[i], b_ref[i])
        if i < 2:
            h = jnp.where(h > 0, h, 0.01 * h)
    o_ref[...] = h


def _dec_call(x, W, b):
    return pl.pallas_call(
        _dec_body,
        grid=(_GRID,),
        in_specs=[
            pl.BlockSpec((_BLK, _D), lambda i: (i, 0)),
            pl.BlockSpec((3, _D, _D), lambda i: (0, 0, 0)),
            pl.BlockSpec((3, _D), lambda i: (0, 0)),
        ],
        out_specs=pl.BlockSpec((_BLK, _D), lambda i: (i, 0)),
        out_shape=jax.ShapeDtypeStruct((_N, _D), jnp.float32),
    )(x, W, b)


def _gat_sc_body(xl_hbm, xr_hbm, idx2_hbm, ea_hbm, wt_hbm, att_hbm,
                 out_hbm, den_hbm,
                 srcc_v, dstc_v, ea_v, xj_v, xi_v,
                 wt_v, att_v, zrow_v, ct_v, tr_v, wbuf_v, den_v,
                 acc_sh, sem0, sem1):
    cid = lax.axis_index("c")
    sid = lax.axis_index("s")
    wid = cid * _NS + sid

    pltpu.sync_copy(wt_hbm, wt_v)
    pltpu.sync_copy(att_hbm, att_v)

    zero16 = jnp.zeros((16,), jnp.float32)

    def _zrow(r, carry):
        def _zcol(q, carry2):
            zrow_v[r, pl.ds(q * 16, 16)] = zero16
            return carry2
        return lax.fori_loop(0, _D // 16, _zcol, carry)

    lax.fori_loop(0, _ZR, _zrow, 0)

    def _zcopy(i, carry):
        pltpu.sync_copy(zrow_v, acc_sh.at[pl.ds(sid * _RPS + i * _ZR, _ZR)])
        return carry

    lax.fori_loop(0, _RPS // _ZR, _zcopy, 0)

    def _zden(i, carry):
        den_v[pl.ds(i * 16, 16)] = zero16
        return carry

    lax.fori_loop(0, _N // 16, _zden, 0)
    plsc.subcore_barrier()

    i16 = lax.iota(jnp.int32, 16)
    basej = [(i16 + j * 16) * 81 for j in range(8)]
    i17 = i16 * 17

    def _chunk(c, carry):
        ch = wid * _NCH + c
        pltpu.sync_copy(idx2_hbm.at[pl.ds(ch * 2 * _CPT, _CPT)], srcc_v)
        pltpu.sync_copy(idx2_hbm.at[pl.ds(ch * 2 * _CPT + _CPT, _CPT)],
                        dstc_v)
        pltpu.sync_copy(ea_hbm.at[pl.ds(ch * _DE * _CPT, _DE * _CPT)], ea_v)
        cpj = pltpu.async_copy(xl_hbm.at[srcc_v], xj_v, sem0)
        cpi = pltpu.async_copy(xr_hbm.at[dstc_v], xi_v, sem1)
        ea_g = [[ea_v[pl.ds(k * _CPT + g * 16, 16)] for k in range(_DE)]
                for g in range(_NG)]

        def _tblock(db, carry2):
            wtb = [wt_v[k, pl.ds(db * 16, 16)] for k in range(_DE)]

            def _td(dd, carry3):
                dsp = jnp.full((16,), dd, jnp.int32)
                wk = [wtb[k].at[dsp].get(mode="promise_in_bounds")
                      for k in range(_DE)]
                d81 = (db * 16 + dd) * 81
                for g in range(_NG):
                    cc = (ea_g[g][0] * wk[0] + ea_g[g][1] * wk[1]
                          + ea_g[g][2] * wk[2] + ea_g[g][3] * wk[3])
                    ct_v[pl.ds(d81 + g * 16, 16)] = cc
                return carry3

            return lax.fori_loop(0, 16, _td, carry2)

        lax.fori_loop(0, 8, _tblock, 0)
        attj = [att_v[pl.ds(j * 16, 16)] for j in range(8)]
        cpj.wait()
        cpi.wait()

        def _group(g, carry2):
            def _edge(ee, carry3):
                e = g * 16 + ee
                esp = jnp.full((16,), e, jnp.int32)
                acc = jnp.zeros((16,), jnp.float32)
                for j in range(8):
                    xjj = xj_v[e, pl.ds(j * 16, 16)]
                    xii = xi_v[e, pl.ds(j * 16, 16)]
                    cj = plsc.load_gather(ct_v, [basej[j] + esp])
                    z = xjj + xii + cj
                    lz = jnp.maximum(z, 0.2 * z)
                    acc = acc + attj[j] * lz
                tr_v[pl.ds(ee * 17, 16)] = acc
                return carry3

            lax.fori_loop(0, 16, _edge, 0)
            tot = jnp.zeros((16,), jnp.float32)
            for j in range(16):
                tot = tot + plsc.load_gather(tr_v, [i17 + j])
            w = jnp.exp(tot)
            wbuf_v[pl.ds(g * 16, 16)] = w
            dstv = dstc_v[pl.ds(g * 16, 16)]
            plsc.addupdate_scatter(den_v, [dstv], w)
            return carry2

        lax.fori_loop(0, _NG, _group, 0)

        def _sgroup(g, carry2):
            w_g = wbuf_v[pl.ds(g * 16, 16)]

            def _sedge(ee, carry3):
                e = g * 16 + ee
                esp = jnp.full((16,), ee, jnp.int32)
                ws = w_g.at[esp].get(mode="promise_in_bounds")
                for j in range(8):
                    v = xj_v[e, pl.ds(j * 16, 16)]
                    xj_v[e, pl.ds(j * 16, 16)] = v * ws
                return carry3

            return lax.fori_loop(0, 16, _sedge, carry2)

        lax.fori_loop(0, _NG, _sgroup, 0)

        pltpu.sync_copy(xj_v, acc_sh.at[dstc_v], add=True)
        return carry

    lax.fori_loop(0, _NCH, _chunk, 0)
    plsc.subcore_barrier()

    pltpu.sync_copy(acc_sh.at[pl.ds(sid * _RPS, _RPS)],
                    out_hbm.at[cid, pl.ds(sid * _RPS, _RPS)])
    pltpu.sync_copy(den_v, den_hbm.at[pl.ds(wid * _N, _N)])


def _gat_sc(xl, xr, idx2, eaflat, wt, att):
    mesh = plsc.VectorSubcoreMesh(core_axis_name="c", subcore_axis_name="s")
    return pl.kernel(
        _gat_sc_body,
        out_type=[
            jax.ShapeDtypeStruct((_NC, _NPAD, _D), jnp.float32),
            jax.ShapeDtypeStruct((_NW * _N,), jnp.float32),
        ],
        mesh=mesh,
        compiler_params=pltpu.CompilerParams(needs_layout_passes=False),
        scratch_types=[
            pltpu.VMEM((_CPT,), jnp.int32),
            pltpu.VMEM((_CPT,), jnp.int32),
            pltpu.VMEM((_DE * _CPT,), jnp.float32),
            pltpu.VMEM((_CPT, _D), jnp.float32),
            pltpu.VMEM((_CPT, _D), jnp.float32),
            pltpu.VMEM((_DE, _D), jnp.float32),
            pltpu.VMEM((_D,), jnp.float32),
            pltpu.VMEM((_ZR, _D), jnp.float32),
            pltpu.VMEM((128 * 81,), jnp.float32),
            pltpu.VMEM((16 * 17,), jnp.float32),
            pltpu.VMEM((_CPT,), jnp.float32),
            pltpu.VMEM((_N,), jnp.float32),
            pltpu.VMEM_SHARED((_NPAD, _D), jnp.float32),
            pltpu.SemaphoreType.DMA,
            pltpu.SemaphoreType.DMA,
        ],
    )(xl, xr, idx2, eaflat, wt, att)


def kernel(x, edge_attr, enc_W, enc_b, enc_ln_g, enc_ln_b, gat_Wl, gat_bl,
           gat_Wr, gat_br, gat_We, gat_att, gat_bias, gln_g, gln_b, dec_W,
           dec_b, edge_index):
    src = edge_index[0].astype(jnp.int32)
    dst = edge_index[1].astype(jnp.int32)
    idx2 = jnp.stack([src.reshape(-1, _CPT), dst.reshape(-1, _CPT)],
                     axis=1).reshape(-1)
    eaflat = jnp.transpose(edge_attr.T.reshape(_DE, -1, _CPT),
                           (1, 0, 2)).reshape(-1)

    y = _enc_call(x, enc_W, enc_b,
                  enc_ln_g.reshape(1, _D), enc_ln_b.reshape(1, _D))
    for i in range(_NB):
        xl, xr = _prep_call(y, gat_Wl[i], gat_bl[i].reshape(1, _D),
                            gat_Wr[i], gat_br[i].reshape(1, _D))
        part, den_flat = _gat_sc(xl, xr, idx2, eaflat,
                                 gat_We[i].T, gat_att[i])
        den_t = den_flat.reshape(_NW, _N).T
        y = _post_call(y, part[0], part[1], den_t,
                       gat_bias[i].reshape(1, _D),
                       gln_g[i].reshape(1, _D), gln_b[i].reshape(1, _D))
    return _dec_call(y, dec_W, dec_b)


# speedup vs baseline: 1.1461x; 1.1461x over previous
import jax
import jax.numpy as jnp
from jax import lax
from jax.experimental import pallas as pl
from jax.experimental.pallas import tpu as pltpu
from jax.experimental.pallas import tpu_sc as plsc

_N = 10000
_E = 320000
_D = 128
_DE = 4
_NB = 3
_NC = 2
_NS = 16
_NW = _NC * _NS
_EPW = _E // _NW
_CPT = 80
_NCH = _EPW // _CPT
_NG = _CPT // 16
_NPAD = 10240
_RPS = _NPAD // _NS
_ZR = 8

_BLK = 400
_GRID = _N // _BLK


def _matTb(h, W, b):
    return lax.dot_general(h, W, (((1,), (1,)), ((), ())),
                           preferred_element_type=jnp.float32) + b


def _layernorm(h, g, b):
    mu = jnp.mean(h, axis=-1, keepdims=True)
    var = jnp.mean((h - mu) ** 2, axis=-1, keepdims=True)
    return (h - mu) / jnp.sqrt(var + 1e-5) * g + b


def _enc_body(x_ref, W_ref, b_ref, g_ref, bb_ref, o_ref):
    h = x_ref[...]
    for i in range(3):
        h = _matTb(h, W_ref[i], b_ref[i])
        if i < 2:
            h = jnp.where(h > 0, h, 0.01 * h)
    o_ref[...] = _layernorm(h, g_ref[...], bb_ref[...])


def _enc_call(x, W, b, g, bb):
    return pl.pallas_call(
        _enc_body,
        grid=(_GRID,),
        in_specs=[
            pl.BlockSpec((_BLK, _D), lambda i: (i, 0)),
            pl.BlockSpec((3, _D, _D), lambda i: (0, 0, 0)),
            pl.BlockSpec((3, _D), lambda i: (0, 0)),
            pl.BlockSpec((1, _D), lambda i: (0, 0)),
            pl.BlockSpec((1, _D), lambda i: (0, 0)),
        ],
        out_specs=pl.BlockSpec((_BLK, _D), lambda i: (i, 0)),
        out_shape=jax.ShapeDtypeStruct((_N, _D), jnp.float32),
    )(x, W, b, g, bb)


def _prep_body(y_ref, Wl_ref, bl_ref, Wr_ref, br_ref, xl_ref, xr_ref):
    y = y_ref[...]
    xl_ref[...] = _matTb(y, Wl_ref[...], bl_ref[...])
    xr_ref[...] = _matTb(y, Wr_ref[...], br_ref[...])


def _prep_call(y, Wl, bl, Wr, br):
    return pl.pallas_call(
        _prep_body,
        grid=(_GRID,),
        in_specs=[
            pl.BlockSpec((_BLK, _D), lambda i: (i, 0)),
            pl.BlockSpec((_D, _D), lambda i: (0, 0)),
            pl.BlockSpec((1, _D), lambda i: (0, 0)),
            pl.BlockSpec((_D, _D), lambda i: (0, 0)),
            pl.BlockSpec((1, _D), lambda i: (0, 0)),
        ],
        out_specs=[
            pl.BlockSpec((_BLK, _D), lambda i: (i, 0)),
            pl.BlockSpec((_BLK, _D), lambda i: (i, 0)),
        ],
        out_shape=[
            jax.ShapeDtypeStruct((_N, _D), jnp.float32),
            jax.ShapeDtypeStruct((_N, _D), jnp.float32),
        ],
    )(y, Wl, bl, Wr, br)


def _post_body(y_ref, p0_ref, p1_ref, den_ref, bias_ref, g_ref, b_ref,
               o_ref):
    num = p0_ref[...] + p1_ref[...]
    den = jnp.sum(den_ref[...], axis=1, keepdims=True)
    gat = num / (den + 1e-16) + bias_ref[...]
    o_ref[...] = y_ref[...] + _layernorm(gat, g_ref[...], b_ref[...])


def _post_call(y, p0, p1, den_t, bias, g, b):
    return pl.pallas_call(
        _post_body,
        grid=(_GRID,),
        in_specs=[
            pl.BlockSpec((_BLK, _D), lambda i: (i, 0)),
            pl.BlockSpec((_BLK, _D), lambda i: (i, 0)),
            pl.BlockSpec((_BLK, _D), lambda i: (i, 0)),
            pl.BlockSpec((_BLK, _NW), lambda i: (i, 0)),
            pl.BlockSpec((1, _D), lambda i: (0, 0)),
            pl.BlockSpec((1, _D), lambda i: (0, 0)),
            pl.BlockSpec((1, _D), lambda i: (0, 0)),
        ],
        out_specs=pl.BlockSpec((_BLK, _D), lambda i: (i, 0)),
        out_shape=jax.ShapeDtypeStruct((_N, _D), jnp.float32),
    )(y, p0, p1, den_t, bias, g, b)


def _dec_body(x_ref, W_ref, b_ref, o_ref):
    h = x_ref[...]
    for i in range(3):
        h = _matTb(h, W_ref[i], b_ref[i])
        if i < 2:
            h = jnp.where(h > 0, h, 0.01 * h)
    o_ref[...] = h


def _dec_call(x, W, b):
    return pl.pallas_call(
        _dec_body,
        grid=(_GRID,),
        in_specs=[
            pl.BlockSpec((_BLK, _D), lambda i: (i, 0)),
            pl.BlockSpec((3, _D, _D), lambda i: (0, 0, 0)),
            pl.BlockSpec((3, _D), lambda i: (0, 0)),
        ],
        out_specs=pl.BlockSpec((_BLK, _D), lambda i: (i, 0)),
        out_shape=jax.ShapeDtypeStruct((_N, _D), jnp.float32),
    )(x, W, b)


def _gat_sc_body(xl_hbm, xr_hbm, idx2_hbm, ea_hbm, wt_hbm, att_hbm,
                 out_hbm, den_hbm,
                 srcA_v, dstA_v, eaA_v, srcB_v, dstB_v, eaB_v,
                 xjA_v, xjB_v, xi_v, dstS_v,
                 wt_v, att_v, zrow_v, tr_v, wbuf_v, den_v,
                 acc_sh, sem0, sem1, semP, semS):
    cid = lax.axis_index("c")
    sid = lax.axis_index("s")
    wid = cid * _NS + sid

    pltpu.sync_copy(wt_hbm, wt_v)
    pltpu.sync_copy(att_hbm, att_v)

    zero16 = jnp.zeros((16,), jnp.float32)

    def _zrow(r, carry):
        def _zcol(q, carry2):
            zrow_v[r, pl.ds(q * 16, 16)] = zero16
            return carry2
        return lax.fori_loop(0, _D // 16, _zcol, carry)

    lax.fori_loop(0, _ZR, _zrow, 0)

    def _zcopy(i, carry):
        pltpu.sync_copy(zrow_v, acc_sh.at[pl.ds(sid * _RPS + i * _ZR, _ZR)])
        return carry

    lax.fori_loop(0, _RPS // _ZR, _zcopy, 0)

    def _zden(i, carry):
        den_v[pl.ds(i * 16, 16)] = zero16
        return carry

    lax.fori_loop(0, _N // 16, _zden, 0)
    plsc.subcore_barrier()

    i16 = lax.iota(jnp.int32, 16)
    i17 = i16 * 17
    attj = [att_v[pl.ds(j * 16, 16)] for j in range(8)]
    wtj = [[wt_v[k, pl.ds(j * 16, 16)] for j in range(8)]
           for k in range(_DE)]

    def _prefetch(c, sv, dv, ev):
        ch = wid * _NCH + c
        o = ch * 2 * _CPT
        pltpu.async_copy(idx2_hbm.at[pl.ds(o, _CPT)], sv, semP)
        pltpu.async_copy(idx2_hbm.at[pl.ds(o + _CPT, _CPT)], dv, semP)
        pltpu.async_copy(ea_hbm.at[pl.ds(ch * _DE * _CPT, _DE * _CPT)],
                         ev, semP)

    def _wait_prefetch(c, sv, dv, ev):
        ch = wid * _NCH + c
        o = ch * 2 * _CPT
        pltpu.make_async_copy(idx2_hbm.at[pl.ds(o, _CPT)], sv, semP).wait()
        pltpu.make_async_copy(idx2_hbm.at[pl.ds(o + _CPT, _CPT)], dv,
                              semP).wait()
        pltpu.make_async_copy(ea_hbm.at[pl.ds(ch * _DE * _CPT,
                                              _DE * _CPT)], ev, semP).wait()

    def _half(c, cur, nxt, first, gather_next, prefetch2):
        srcc_v, dstc_v, ea_v, xj_v = cur
        pltpu.make_async_copy(xl_hbm.at[srcc_v], xj_v, sem0).wait()
        pltpu.make_async_copy(xr_hbm.at[dstc_v], xi_v, sem1).wait()

        def _group(g, carry2):
            ea_vec = [ea_v[pl.ds(k * _CPT + g * 16, 16)]
                      for k in range(_DE)]

            def _edge(ee, carry3):
                e = g * 16 + ee
                esp = jnp.full((16,), ee, jnp.int32)
                eas = [ea_vec[k].at[esp].get(mode="promise_in_bounds")
                       for k in range(_DE)]
                acc = jnp.zeros((16,), jnp.float32)
                for j in range(8):
                    z = (xj_v[e, pl.ds(j * 16, 16)]
                         + xi_v[e, pl.ds(j * 16, 16)]
                         + (eas[0] * wtj[0][j] + eas[1] * wtj[1][j])
                         + (eas[2] * wtj[2][j] + eas[3] * wtj[3][j]))
                    lz = jnp.maximum(z, 0.2 * z)
                    acc = acc + attj[j] * lz
                tr_v[pl.ds(ee * 17, 16)] = acc
                return carry3

            lax.fori_loop(0, 16, _edge, 0)
            tot = jnp.zeros((16,), jnp.float32)
            for j in range(16):
                tot = tot + plsc.load_gather(tr_v, [i17 + j])
            w = jnp.exp(tot)
            wbuf_v[pl.ds(g * 16, 16)] = w
            dstv = dstc_v[pl.ds(g * 16, 16)]
            plsc.addupdate_scatter(den_v, [dstv], w)
            return carry2

        lax.fori_loop(0, _NG, _group, 0)

        if not first:
            pltpu.make_async_copy(nxt[3], acc_sh.at[dstS_v], semS).wait()

        def _snap(g, carry2):
            dstS_v[pl.ds(g * 16, 16)] = dstc_v[pl.ds(g * 16, 16)]
            return carry2

        lax.fori_loop(0, _NG, _snap, 0)

        if gather_next:
            srcn_v, dstn_v, ean_v, xjn_v = nxt
            _wait_prefetch(c + 1, srcn_v, dstn_v, ean_v)
            pltpu.async_copy(xl_hbm.at[srcn_v], xjn_v, sem0)
            pltpu.async_copy(xr_hbm.at[dstn_v], xi_v, sem1)
            if prefetch2:
                _prefetch(c + 2, srcc_v, dstc_v, ea_v)

        def _sgroup(g, carry2):
            w_g = wbuf_v[pl.ds(g * 16, 16)]

            def _sedge(ee, carry3):
                e = g * 16 + ee
                esp = jnp.full((16,), ee, jnp.int32)
                ws = w_g.at[esp].get(mode="promise_in_bounds")
                for j in range(8):
                    v = xj_v[e, pl.ds(j * 16, 16)]
                    xj_v[e, pl.ds(j * 16, 16)] = v * ws
                return carry3

            return lax.fori_loop(0, 16, _sedge, carry2)

        lax.fori_loop(0, _NG, _sgroup, 0)

        pltpu.async_copy(xj_v, acc_sh.at[dstS_v], semS, add=True)

    bufA = (srcA_v, dstA_v, eaA_v, xjA_v)
    bufB = (srcB_v, dstB_v, eaB_v, xjB_v)

    _prefetch(0, srcA_v, dstA_v, eaA_v)
    _wait_prefetch(0, srcA_v, dstA_v, eaA_v)
    pltpu.async_copy(xl_hbm.at[srcA_v], xjA_v, sem0)
    pltpu.async_copy(xr_hbm.at[dstA_v], xi_v, sem1)
    _prefetch(1, srcB_v, dstB_v, eaB_v)

    _half(0, bufA, bufB, first=True, gather_next=True, prefetch2=True)

    def _pair(h, carry):
        c0 = 1 + 2 * h
        _half(c0, bufB, bufA, first=False, gather_next=True, prefetch2=True)
        _half(c0 + 1, bufA, bufB, first=False, gather_next=True,
              prefetch2=True)
        return carry

    lax.fori_loop(0, (_NCH - 3) // 2, _pair, 0)
    _half(_NCH - 2, bufB, bufA, first=False, gather_next=True,
          prefetch2=False)
    _half(_NCH - 1, bufA, bufB, first=False, gather_next=False,
          prefetch2=False)
    pltpu.make_async_copy(xjA_v, acc_sh.at[dstS_v], semS).wait()

    plsc.subcore_barrier()

    pltpu.sync_copy(acc_sh.at[pl.ds(sid * _RPS, _RPS)],
                    out_hbm.at[cid, pl.ds(sid * _RPS, _RPS)])
    pltpu.sync_copy(den_v, den_hbm.at[pl.ds(wid * _N, _N)])


def _gat_sc(xl, xr, idx2, eaflat, wt, att):
    mesh = plsc.VectorSubcoreMesh(core_axis_name="c", subcore_axis_name="s")
    return pl.kernel(
        _gat_sc_body,
        out_type=[
            jax.ShapeDtypeStruct((_NC, _NPAD, _D), jnp.float32),
            jax.ShapeDtypeStruct((_NW * _N,), jnp.float32),
        ],
        mesh=mesh,
        compiler_params=pltpu.CompilerParams(needs_layout_passes=False),
        scratch_types=[
            pltpu.VMEM((_CPT,), jnp.int32),
            pltpu.VMEM((_CPT,), jnp.int32),
            pltpu.VMEM((_DE * _CPT,), jnp.float32),
            pltpu.VMEM((_CPT,), jnp.int32),
            pltpu.VMEM((_CPT,), jnp.int32),
            pltpu.VMEM((_DE * _CPT,), jnp.float32),
            pltpu.VMEM((_CPT, _D), jnp.float32),
            pltpu.VMEM((_CPT, _D), jnp.float32),
            pltpu.VMEM((_CPT, _D), jnp.float32),
            pltpu.VMEM((_CPT,), jnp.int32),
            pltpu.VMEM((_DE, _D), jnp.float32),
            pltpu.VMEM((_D,), jnp.float32),
            pltpu.VMEM((_ZR, _D), jnp.float32),
            pltpu.VMEM((16 * 17,), jnp.float32),
            pltpu.VMEM((_CPT,), jnp.float32),
            pltpu.VMEM((_N,), jnp.float32),
            pltpu.VMEM_SHARED((_NPAD, _D), jnp.float32),
            pltpu.SemaphoreType.DMA,
            pltpu.SemaphoreType.DMA,
            pltpu.SemaphoreType.DMA,
            pltpu.SemaphoreType.DMA,
        ],
    )(xl, xr, idx2, eaflat, wt, att)


def kernel(x, edge_attr, enc_W, enc_b, enc_ln_g, enc_ln_b, gat_Wl, gat_bl,
           gat_Wr, gat_br, gat_We, gat_att, gat_bias, gln_g, gln_b, dec_W,
           dec_b, edge_index):
    src = edge_index[0].astype(jnp.int32)
    dst = edge_index[1].astype(jnp.int32)
    idx2 = jnp.stack([src.reshape(-1, _CPT), dst.reshape(-1, _CPT)],
                     axis=1).reshape(-1)
    eaflat = jnp.transpose(edge_attr.T.reshape(_DE, -1, _CPT),
                           (1, 0, 2)).reshape(-1)

    y = _enc_call(x, enc_W, enc_b,
                  enc_ln_g.reshape(1, _D), enc_ln_b.reshape(1, _D))
    for i in range(_NB):
        xl, xr = _prep_call(y, gat_Wl[i], gat_bl[i].reshape(1, _D),
                            gat_Wr[i], gat_br[i].reshape(1, _D))
        part, den_flat = _gat_sc(xl, xr, idx2, eaflat,
                                 gat_We[i].T, gat_att[i])
        den_t = den_flat.reshape(_NW, _N).T
        y = _post_call(y, part[0], part[1], den_t,
                       gat_bias[i].reshape(1, _D),
                       gln_g[i].reshape(1, _D), gln_b[i].reshape(1, _D))
    return _dec_call(y, dec_W, dec_b)


# speedup vs baseline: 1.1554x; 1.0082x over previous
import jax
import jax.numpy as jnp
from jax import lax
from jax.experimental import pallas as pl
from jax.experimental.pallas import tpu as pltpu
from jax.experimental.pallas import tpu_sc as plsc

_N = 10000
_E = 320000
_D = 128
_DE = 4
_NB = 3
_NC = 2
_NS = 16
_NW = _NC * _NS
_EPW = _E // _NW
_CPT = 80
_NCH = _EPW // _CPT
_NG = _CPT // 16
_NPAD = 10240
_RPS = _NPAD // _NS
_ZR = 8

_BLK = 400
_GRID = _N // _BLK


def _matTb(h, W, b):
    return lax.dot_general(h, W, (((1,), (1,)), ((), ())),
                           preferred_element_type=jnp.float32) + b


def _layernorm(h, g, b):
    mu = jnp.mean(h, axis=-1, keepdims=True)
    var = jnp.mean((h - mu) ** 2, axis=-1, keepdims=True)
    return (h - mu) / jnp.sqrt(var + 1e-5) * g + b


def _enc_body(x_ref, W_ref, b_ref, g_ref, bb_ref, o_ref):
    h = x_ref[...]
    for i in range(3):
        h = _matTb(h, W_ref[i], b_ref[i])
        if i < 2:
            h = jnp.where(h > 0, h, 0.01 * h)
    o_ref[...] = _layernorm(h, g_ref[...], bb_ref[...])


def _enc_call(x, W, b, g, bb):
    return pl.pallas_call(
        _enc_body,
        grid=(_GRID,),
        in_specs=[
            pl.BlockSpec((_BLK, _D), lambda i: (i, 0)),
            pl.BlockSpec((3, _D, _D), lambda i: (0, 0, 0)),
            pl.BlockSpec((3, _D), lambda i: (0, 0)),
            pl.BlockSpec((1, _D), lambda i: (0, 0)),
            pl.BlockSpec((1, _D), lambda i: (0, 0)),
        ],
        out_specs=pl.BlockSpec((_BLK, _D), lambda i: (i, 0)),
        out_shape=jax.ShapeDtypeStruct((_N, _D), jnp.float32),
    )(x, W, b, g, bb)


def _prep_body(y_ref, Wl_ref, bl_ref, Wr_ref, br_ref, xl_ref, xr_ref):
    y = y_ref[...]
    xl_ref[...] = _matTb(y, Wl_ref[...], bl_ref[...])
    xr_ref[...] = _matTb(y, Wr_ref[...], br_ref[...])


def _prep_call(y, Wl, bl, Wr, br):
    return pl.pallas_call(
        _prep_body,
        grid=(_GRID,),
        in_specs=[
            pl.BlockSpec((_BLK, _D), lambda i: (i, 0)),
            pl.BlockSpec((_D, _D), lambda i: (0, 0)),
            pl.BlockSpec((1, _D), lambda i: (0, 0)),
            pl.BlockSpec((_D, _D), lambda i: (0, 0)),
            pl.BlockSpec((1, _D), lambda i: (0, 0)),
        ],
        out_specs=[
            pl.BlockSpec((_BLK, _D), lambda i: (i, 0)),
            pl.BlockSpec((_BLK, _D), lambda i: (i, 0)),
        ],
        out_shape=[
            jax.ShapeDtypeStruct((_N, _D), jnp.float32),
            jax.ShapeDtypeStruct((_N, _D), jnp.float32),
        ],
    )(y, Wl, bl, Wr, br)


def _post_body(y_ref, p0_ref, p1_ref, den_ref, bias_ref, g_ref, b_ref,
               o_ref):
    num = p0_ref[...] + p1_ref[...]
    den = jnp.sum(den_ref[...], axis=1, keepdims=True)
    gat = num / (den + 1e-16) + bias_ref[...]
    o_ref[...] = y_ref[...] + _layernorm(gat, g_ref[...], b_ref[...])


def _post_call(y, p0, p1, den_t, bias, g, b):
    return pl.pallas_call(
        _post_body,
        grid=(_GRID,),
        in_specs=[
            pl.BlockSpec((_BLK, _D), lambda i: (i, 0)),
            pl.BlockSpec((_BLK, _D), lambda i: (i, 0)),
            pl.BlockSpec((_BLK, _D), lambda i: (i, 0)),
            pl.BlockSpec((_BLK, _NW), lambda i: (i, 0)),
            pl.BlockSpec((1, _D), lambda i: (0, 0)),
            pl.BlockSpec((1, _D), lambda i: (0, 0)),
            pl.BlockSpec((1, _D), lambda i: (0, 0)),
        ],
        out_specs=pl.BlockSpec((_BLK, _D), lambda i: (i, 0)),
        out_shape=jax.ShapeDtypeStruct((_N, _D), jnp.float32),
    )(y, p0, p1, den_t, bias, g, b)


def _dec_body(x_ref, W_ref, b_ref, o_ref):
    h = x_ref[...]
    for i in range(3):
        h = _matTb(h, W_ref[i], b_ref[i])
        if i < 2:
            h = jnp.where(h > 0, h, 0.01 * h)
    o_ref[...] = h


def _dec_call(x, W, b):
    return pl.pallas_call(
        _dec_body,
        grid=(_GRID,),
        in_specs=[
            pl.BlockSpec((_BLK, _D), lambda i: (i, 0)),
            pl.BlockSpec((3, _D, _D), lambda i: (0, 0, 0)),
            pl.BlockSpec((3, _D), lambda i: (0, 0)),
        ],
        out_specs=pl.BlockSpec((_BLK, _D), lambda i: (i, 0)),
        out_shape=jax.ShapeDtypeStruct((_N, _D), jnp.float32),
    )(x, W, b)


def _gat_sc_body(xl_hbm, xr_hbm, idx2_hbm, ea_hbm, wt_hbm, att_hbm,
                 out_hbm, den_hbm,
                 srcA_v, dstA_v, eaA_v, srcB_v, dstB_v, eaB_v,
                 xjA_v, xjB_v, xi_v, dstS_v,
                 wt_v, att_v, zrow_v, ct_v, tr_v, wbuf_v, den_v,
                 acc_sh, sem0, sem1, semP, semS):
    cid = lax.axis_index("c")
    sid = lax.axis_index("s")
    wid = cid * _NS + sid

    pltpu.sync_copy(wt_hbm, wt_v)
    pltpu.sync_copy(att_hbm, att_v)

    zero16 = jnp.zeros((16,), jnp.float32)

    def _zrow(r, carry):
        def _zcol(q, carry2):
            zrow_v[r, pl.ds(q * 16, 16)] = zero16
            return carry2
        return lax.fori_loop(0, _D // 16, _zcol, carry)

    lax.fori_loop(0, _ZR, _zrow, 0)

    def _zcopy(i, carry):
        pltpu.sync_copy(zrow_v, acc_sh.at[pl.ds(sid * _RPS + i * _ZR, _ZR)])
        return carry

    lax.fori_loop(0, _RPS // _ZR, _zcopy, 0)

    def _zden(i, carry):
        den_v[pl.ds(i * 16, 16)] = zero16
        return carry

    lax.fori_loop(0, _N // 16, _zden, 0)
    plsc.subcore_barrier()

    i16 = lax.iota(jnp.int32, 16)
    i17 = i16 * 17
    basej33 = [(i16 + j * 16) * 33 for j in range(8)]
    attj = [att_v[pl.ds(j * 16, 16)] for j in range(8)]

    def _prefetch(c, sv, dv, ev):
        ch = wid * _NCH + c
        o = ch * 2 * _CPT
        pltpu.async_copy(idx2_hbm.at[pl.ds(o, _CPT)], sv, semP)
        pltpu.async_copy(idx2_hbm.at[pl.ds(o + _CPT, _CPT)], dv, semP)
        pltpu.async_copy(ea_hbm.at[pl.ds(ch * _DE * _CPT, _DE * _CPT)],
                         ev, semP)

    def _wait_prefetch(c, sv, dv, ev):
        ch = wid * _NCH + c
        o = ch * 2 * _CPT
        pltpu.make_async_copy(idx2_hbm.at[pl.ds(o, _CPT)], sv, semP).wait()
        pltpu.make_async_copy(idx2_hbm.at[pl.ds(o + _CPT, _CPT)], dv,
                              semP).wait()
        pltpu.make_async_copy(ea_hbm.at[pl.ds(ch * _DE * _CPT,
                                              _DE * _CPT)], ev, semP).wait()

    def _half(c, cur, nxt, first, gather_next, prefetch2):
        srcc_v, dstc_v, ea_v, xj_v = cur
        ea_g = [[ea_v[pl.ds(k * _CPT + g * 16, 16)] for k in range(_DE)]
                for g in range(_NG)]

        def _stage1(goff, gcnt):
            def _tblock(db, carry2):
                wtb = [wt_v[k, pl.ds(db * 16, 16)] for k in range(_DE)]

                def _td(dd, carry3):
                    dsp = jnp.full((16,), dd, jnp.int32)
                    wk = [wtb[k].at[dsp].get(mode="promise_in_bounds")
                          for k in range(_DE)]
                    d33 = (db * 16 + dd) * 33
                    for gg in range(gcnt):
                        eg = ea_g[goff + gg]
                        cc = (eg[0] * wk[0] + eg[1] * wk[1]
                              + eg[2] * wk[2] + eg[3] * wk[3])
                        ct_v[pl.ds(d33 + gg * 16, 16)] = cc
                    return carry3

                return lax.fori_loop(0, 16, _td, carry2)

            lax.fori_loop(0, 8, _tblock, 0)
            if goff == 0:
                pltpu.make_async_copy(xl_hbm.at[srcc_v], xj_v, sem0).wait()
                pltpu.make_async_copy(xr_hbm.at[dstc_v], xi_v, sem1).wait()
            for gg in range(gcnt):
                g = goff + gg

                def _edge(ee, carry3):
                    e = g * 16 + ee
                    el = gg * 16 + ee
                    esp = jnp.full((16,), el, jnp.int32)
                    acc = jnp.zeros((16,), jnp.float32)
                    for j in range(8):
                        xjj = xj_v[e, pl.ds(j * 16, 16)]
                        xii = xi_v[e, pl.ds(j * 16, 16)]
                        cj = plsc.load_gather(ct_v, [basej33[j] + esp])
                        z = xjj + xii + cj
                        lz = jnp.maximum(z, 0.2 * z)
                        acc = acc + attj[j] * lz
                    tr_v[pl.ds(ee * 17, 16)] = acc
                    return carry3

                lax.fori_loop(0, 16, _edge, 0)
                tot = jnp.zeros((16,), jnp.float32)
                for j in range(16):
                    tot = tot + plsc.load_gather(tr_v, [i17 + j])
                w = jnp.exp(tot)
                wbuf_v[pl.ds(g * 16, 16)] = w
                dstv = dstc_v[pl.ds(g * 16, 16)]
                plsc.addupdate_scatter(den_v, [dstv], w)

        _stage1(0, 2)
        _stage1(2, 2)
        _stage1(4, 1)

        if not first:
            pltpu.make_async_copy(nxt[3], acc_sh.at[dstS_v], semS).wait()

        def _snap(g, carry2):
            dstS_v[pl.ds(g * 16, 16)] = dstc_v[pl.ds(g * 16, 16)]
            return carry2

        lax.fori_loop(0, _NG, _snap, 0)

        if gather_next:
            srcn_v, dstn_v, ean_v, xjn_v = nxt
            _wait_prefetch(c + 1, srcn_v, dstn_v, ean_v)
            pltpu.async_copy(xl_hbm.at[srcn_v], xjn_v, sem0)
            pltpu.async_copy(xr_hbm.at[dstn_v], xi_v, sem1)
            if prefetch2:
                _prefetch(c + 2, srcc_v, dstc_v, ea_v)

        def _sgroup(g, carry2):
            w_g = wbuf_v[pl.ds(g * 16, 16)]

            def _sedge(ee, carry3):
                e = g * 16 + ee
                esp = jnp.full((16,), ee, jnp.int32)
                ws = w_g.at[esp].get(mode="promise_in_bounds")
                for j in range(8):
                    v = xj_v[e, pl.ds(j * 16, 16)]
                    xj_v[e, pl.ds(j * 16, 16)] = v * ws
                return carry3

            return lax.fori_loop(0, 16, _sedge, carry2)

        lax.fori_loop(0, _NG, _sgroup, 0)

        pltpu.async_copy(xj_v, acc_sh.at[dstS_v], semS, add=True)

    bufA = (srcA_v, dstA_v, eaA_v, xjA_v)
    bufB = (srcB_v, dstB_v, eaB_v, xjB_v)

    _prefetch(0, srcA_v, dstA_v, eaA_v)
    _wait_prefetch(0, srcA_v, dstA_v, eaA_v)
    pltpu.async_copy(xl_hbm.at[srcA_v], xjA_v, sem0)
    pltpu.async_copy(xr_hbm.at[dstA_v], xi_v, sem1)
    _prefetch(1, srcB_v, dstB_v, eaB_v)

    _half(0, bufA, bufB, first=True, gather_next=True, prefetch2=True)

    def _pair(h, carry):
        c0 = 1 + 2 * h
        _half(c0, bufB, bufA, first=False, gather_next=True, prefetch2=True)
        _half(c0 + 1, bufA, bufB, first=False, gather_next=True,
              prefetch2=True)
        return carry

    lax.fori_loop(0, (_NCH - 3) // 2, _pair, 0)
    _half(_NCH - 2, bufB, bufA, first=False, gather_next=True,
          prefetch2=False)
    _half(_NCH - 1, bufA, bufB, first=False, gather_next=False,
          prefetch2=False)
    pltpu.make_async_copy(xjA_v, acc_sh.at[dstS_v], semS).wait()

    plsc.subcore_barrier()

    pltpu.sync_copy(acc_sh.at[pl.ds(sid * _RPS, _RPS)],
                    out_hbm.at[cid, pl.ds(sid * _RPS, _RPS)])
    pltpu.sync_copy(den_v, den_hbm.at[pl.ds(wid * _N, _N)])


def _gat_sc(xl, xr, idx2, eaflat, wt, att):
    mesh = plsc.VectorSubcoreMesh(core_axis_name="c", subcore_axis_name="s")
    return pl.kernel(
        _gat_sc_body,
        out_type=[
            jax.ShapeDtypeStruct((_NC, _NPAD, _D), jnp.float32),
            jax.ShapeDtypeStruct((_NW * _N,), jnp.float32),
        ],
        mesh=mesh,
        compiler_params=pltpu.CompilerParams(needs_layout_passes=False),
        scratch_types=[
            pltpu.VMEM((_CPT,), jnp.int32),
            pltpu.VMEM((_CPT,), jnp.int32),
            pltpu.VMEM((_DE * _CPT,), jnp.float32),
            pltpu.VMEM((_CPT,), jnp.int32),
            pltpu.VMEM((_CPT,), jnp.int32),
            pltpu.VMEM((_DE * _CPT,), jnp.float32),
            pltpu.VMEM((_CPT, _D), jnp.float32),
            pltpu.VMEM((_CPT, _D), jnp.float32),
            pltpu.VMEM((_CPT, _D), jnp.float32),
            pltpu.VMEM((_CPT,), jnp.int32),
            pltpu.VMEM((_DE, _D), jnp.float32),
            pltpu.VMEM((_D,), jnp.float32),
            pltpu.VMEM((_ZR, _D), jnp.float32),
            pltpu.VMEM((128 * 33,), jnp.float32),
            pltpu.VMEM((16 * 17,), jnp.float32),
            pltpu.VMEM((_CPT,), jnp.float32),
            pltpu.VMEM((_N,), jnp.float32),
            pltpu.VMEM_SHARED((_NPAD, _D), jnp.float32),
            pltpu.SemaphoreType.DMA,
            pltpu.SemaphoreType.DMA,
            pltpu.SemaphoreType.DMA,
            pltpu.SemaphoreType.DMA,
        ],
    )(xl, xr, idx2, eaflat, wt, att)


def kernel(x, edge_attr, enc_W, enc_b, enc_ln_g, enc_ln_b, gat_Wl, gat_bl,
           gat_Wr, gat_br, gat_We, gat_att, gat_bias, gln_g, gln_b, dec_W,
           dec_b, edge_index):
    src = edge_index[0].astype(jnp.int32)
    dst = edge_index[1].astype(jnp.int32)
    idx2 = jnp.stack([src.reshape(-1, _CPT), dst.reshape(-1, _CPT)],
                     axis=1).reshape(-1)
    eaflat = jnp.transpose(edge_attr.T.reshape(_DE, -1, _CPT),
                           (1, 0, 2)).reshape(-1)

    y = _enc_call(x, enc_W, enc_b,
                  enc_ln_g.reshape(1, _D), enc_ln_b.reshape(1, _D))
    for i in range(_NB):
        xl, xr = _prep_call(y, gat_Wl[i], gat_bl[i].reshape(1, _D),
                            gat_Wr[i], gat_br[i].reshape(1, _D))
        part, den_flat = _gat_sc(xl, xr, idx2, eaflat,
                                 gat_We[i].T, gat_att[i])
        den_t = den_flat.reshape(_NW, _N).T
        y = _post_call(y, part[0], part[1], den_t,
                       gat_bias[i].reshape(1, _D),
                       gln_g[i].reshape(1, _D), gln_b[i].reshape(1, _D))
    return _dec_call(y, dec_W, dec_b)


# speedup vs baseline: 1.2503x; 1.0821x over previous
import jax
import jax.numpy as jnp
from jax import lax
from jax.experimental import pallas as pl
from jax.experimental.pallas import tpu as pltpu
from jax.experimental.pallas import tpu_sc as plsc

_N = 10000
_E = 320000
_D = 128
_DE = 4
_NB = 3
_NC = 2
_NS = 16
_NW = _NC * _NS
_EPW = _E // _NW
_CPT = 80
_NCH = _EPW // _CPT
_NG = _CPT // 16
_NPAD = 10240
_RPS = _NPAD // _NS
_ZR = 8

_BLK = 400
_GRID = _N // _BLK


def _matTb(h, W, b):
    return lax.dot_general(h, W, (((1,), (1,)), ((), ())),
                           preferred_element_type=jnp.float32) + b


def _layernorm(h, g, b):
    mu = jnp.mean(h, axis=-1, keepdims=True)
    var = jnp.mean((h - mu) ** 2, axis=-1, keepdims=True)
    return (h - mu) / jnp.sqrt(var + 1e-5) * g + b


def _enc_body(x_ref, W_ref, b_ref, g_ref, bb_ref, o_ref):
    h = x_ref[...]
    for i in range(3):
        h = _matTb(h, W_ref[i], b_ref[i])
        if i < 2:
            h = jnp.where(h > 0, h, 0.01 * h)
    o_ref[...] = _layernorm(h, g_ref[...], bb_ref[...])


def _enc_call(x, W, b, g, bb):
    return pl.pallas_call(
        _enc_body,
        grid=(_GRID,),
        in_specs=[
            pl.BlockSpec((_BLK, _D), lambda i: (i, 0)),
            pl.BlockSpec((3, _D, _D), lambda i: (0, 0, 0)),
            pl.BlockSpec((3, _D), lambda i: (0, 0)),
            pl.BlockSpec((1, _D), lambda i: (0, 0)),
            pl.BlockSpec((1, _D), lambda i: (0, 0)),
        ],
        out_specs=pl.BlockSpec((_BLK, _D), lambda i: (i, 0)),
        out_shape=jax.ShapeDtypeStruct((_N, _D), jnp.float32),
    )(x, W, b, g, bb)


def _prep_body(y_ref, Wl_ref, bl_ref, Wr_ref, br_ref, xl_ref, xr_ref):
    y = y_ref[...]
    xl_ref[...] = _matTb(y, Wl_ref[...], bl_ref[...])
    xr_ref[...] = _matTb(y, Wr_ref[...], br_ref[...])


def _prep_call(y, Wl, bl, Wr, br):
    return pl.pallas_call(
        _prep_body,
        grid=(_GRID,),
        in_specs=[
            pl.BlockSpec((_BLK, _D), lambda i: (i, 0)),
            pl.BlockSpec((_D, _D), lambda i: (0, 0)),
            pl.BlockSpec((1, _D), lambda i: (0, 0)),
            pl.BlockSpec((_D, _D), lambda i: (0, 0)),
            pl.BlockSpec((1, _D), lambda i: (0, 0)),
        ],
        out_specs=[
            pl.BlockSpec((_BLK, _D), lambda i: (i, 0)),
            pl.BlockSpec((_BLK, _D), lambda i: (i, 0)),
        ],
        out_shape=[
            jax.ShapeDtypeStruct((_N, _D), jnp.float32),
            jax.ShapeDtypeStruct((_N, _D), jnp.float32),
        ],
    )(y, Wl, bl, Wr, br)


def _post_body(y_ref, p0_ref, p1_ref, den_ref, bias_ref, g_ref, b_ref,
               o_ref):
    num = p0_ref[...] + p1_ref[...]
    den = jnp.sum(den_ref[...], axis=1, keepdims=True)
    gat = num / (den + 1e-16) + bias_ref[...]
    o_ref[...] = y_ref[...] + _layernorm(gat, g_ref[...], b_ref[...])


def _post_call(y, p0, p1, den_t, bias, g, b):
    return pl.pallas_call(
        _post_body,
        grid=(_GRID,),
        in_specs=[
            pl.BlockSpec((_BLK, _D), lambda i: (i, 0)),
            pl.BlockSpec((_BLK, _D), lambda i: (i, 0)),
            pl.BlockSpec((_BLK, _D), lambda i: (i, 0)),
            pl.BlockSpec((_BLK, _NW), lambda i: (i, 0)),
            pl.BlockSpec((1, _D), lambda i: (0, 0)),
            pl.BlockSpec((1, _D), lambda i: (0, 0)),
            pl.BlockSpec((1, _D), lambda i: (0, 0)),
        ],
        out_specs=pl.BlockSpec((_BLK, _D), lambda i: (i, 0)),
        out_shape=jax.ShapeDtypeStruct((_N, _D), jnp.float32),
    )(y, p0, p1, den_t, bias, g, b)


def _dec_body(x_ref, W_ref, b_ref, o_ref):
    h = x_ref[...]
    for i in range(3):
        h = _matTb(h, W_ref[i], b_ref[i])
        if i < 2:
            h = jnp.where(h > 0, h, 0.01 * h)
    o_ref[...] = h


def _dec_call(x, W, b):
    return pl.pallas_call(
        _dec_body,
        grid=(_GRID,),
        in_specs=[
            pl.BlockSpec((_BLK, _D), lambda i: (i, 0)),
            pl.BlockSpec((3, _D, _D), lambda i: (0, 0, 0)),
            pl.BlockSpec((3, _D), lambda i: (0, 0)),
        ],
        out_specs=pl.BlockSpec((_BLK, _D), lambda i: (i, 0)),
        out_shape=jax.ShapeDtypeStruct((_N, _D), jnp.float32),
    )(x, W, b)


def _gat_sc_body(xl_hbm, xr_hbm, idx2_hbm, ea_hbm, wt_hbm, att_hbm,
                 out_hbm, den_hbm,
                 srcA_v, dstA_v, eaA_v, srcB_v, dstB_v, eaB_v, xj_v, xi_v,
                 wt_v, att_v, zrow_v, ct_v, tr_v, wbuf_v, den_v,
                 acc_sh, sem0, sem1, semP, semS):
    cid = lax.axis_index("c")
    sid = lax.axis_index("s")
    wid = cid * _NS + sid

    pltpu.sync_copy(wt_hbm, wt_v)
    pltpu.sync_copy(att_hbm, att_v)

    zero16 = jnp.zeros((16,), jnp.float32)

    def _zrow(r, carry):
        def _zcol(q, carry2):
            zrow_v[r, pl.ds(q * 16, 16)] = zero16
            return carry2
        return lax.fori_loop(0, _D // 16, _zcol, carry)

    lax.fori_loop(0, _ZR, _zrow, 0)

    def _zcopy(i, carry):
        pltpu.sync_copy(zrow_v, acc_sh.at[pl.ds(sid * _RPS + i * _ZR, _ZR)])
        return carry

    lax.fori_loop(0, _RPS // _ZR, _zcopy, 0)

    def _zden(i, carry):
        den_v[pl.ds(i * 16, 16)] = zero16
        return carry

    lax.fori_loop(0, _N // 16, _zden, 0)
    plsc.subcore_barrier()

    i16 = lax.iota(jnp.int32, 16)
    basej = [(i16 + j * 16) * 81 for j in range(8)]
    i17 = i16 * 17
    attj = [att_v[pl.ds(j * 16, 16)] for j in range(8)]

    def _prefetch(c, sv, dv, ev):
        ch = wid * _NCH + c
        o = ch * 2 * _CPT
        pltpu.async_copy(idx2_hbm.at[pl.ds(o, _CPT)], sv, semP)
        pltpu.async_copy(idx2_hbm.at[pl.ds(o + _CPT, _CPT)], dv, semP)
        pltpu.async_copy(ea_hbm.at[pl.ds(ch * _DE * _CPT, _DE * _CPT)],
                         ev, semP)

    def _wait_prefetch(c, sv, dv, ev):
        ch = wid * _NCH + c
        o = ch * 2 * _CPT
        pltpu.make_async_copy(idx2_hbm.at[pl.ds(o, _CPT)], sv, semP).wait()
        pltpu.make_async_copy(idx2_hbm.at[pl.ds(o + _CPT, _CPT)], dv,
                              semP).wait()
        pltpu.make_async_copy(ea_hbm.at[pl.ds(ch * _DE * _CPT,
                                              _DE * _CPT)], ev, semP).wait()

    def _half(c, cur, nxt, have_prev, do_prefetch):
        srcc_v, dstc_v, ea_v = cur
        _wait_prefetch(c, srcc_v, dstc_v, ea_v)
        if have_prev:
            pltpu.make_async_copy(xj_v, acc_sh.at[dstc_v], semS).wait()
        cpj = pltpu.async_copy(xl_hbm.at[srcc_v], xj_v, sem0)
        cpi = pltpu.async_copy(xr_hbm.at[dstc_v], xi_v, sem1)
        if do_prefetch:
            _prefetch(c + 1, *nxt)
        ea_g = [[ea_v[pl.ds(k * _CPT + g * 16, 16)] for k in range(_DE)]
                for g in range(_NG)]

        def _tblock(db, carry2):
            wtb = [wt_v[k, pl.ds(db * 16, 16)] for k in range(_DE)]

            def _td(dd, carry3):
                dsp = jnp.full((16,), dd, jnp.int32)
                wk = [wtb[k].at[dsp].get(mode="promise_in_bounds")
                      for k in range(_DE)]
                d81 = (db * 16 + dd) * 81
                for g in range(_NG):
                    cc = (ea_g[g][0] * wk[0] + ea_g[g][1] * wk[1]
                          + ea_g[g][2] * wk[2] + ea_g[g][3] * wk[3])
                    ct_v[pl.ds(d81 + g * 16, 16)] = cc
                return carry3

            return lax.fori_loop(0, 16, _td, carry2)

        lax.fori_loop(0, 8, _tblock, 0)
        cpj.wait()
        cpi.wait()

        def _group(g, carry2):
            def _edge(ee, carry3):
                e = g * 16 + ee
                esp = jnp.full((16,), e, jnp.int32)
                acc = jnp.zeros((16,), jnp.float32)
                for j in range(8):
                    xjj = xj_v[e, pl.ds(j * 16, 16)]
                    xii = xi_v[e, pl.ds(j * 16, 16)]
                    cj = plsc.load_gather(ct_v, [basej[j] + esp])
                    z = xjj + xii + cj
                    lz = jnp.maximum(z, 0.2 * z)
                    acc = acc + attj[j] * lz
                tr_v[pl.ds(ee * 17, 16)] = acc
                return carry3

            lax.fori_loop(0, 16, _edge, 0)
            tot = jnp.zeros((16,), jnp.float32)
            for j in range(16):
                tot = tot + plsc.load_gather(tr_v, [i17 + j])
            w = jnp.exp(tot)
            wbuf_v[pl.ds(g * 16, 16)] = w
            dstv = dstc_v[pl.ds(g * 16, 16)]
            plsc.addupdate_scatter(den_v, [dstv], w)
            return carry2

        lax.fori_loop(0, _NG, _group, 0)

        def _sgroup(g, carry2):
            w_g = wbuf_v[pl.ds(g * 16, 16)]

            def _sedge(ee, carry3):
                e = g * 16 + ee
                esp = jnp.full((16,), ee, jnp.int32)
                ws = w_g.at[esp].get(mode="promise_in_bounds")
                for j in range(8):
                    v = xj_v[e, pl.ds(j * 16, 16)]
                    xj_v[e, pl.ds(j * 16, 16)] = v * ws
                return carry3

            return lax.fori_loop(0, 16, _sedge, carry2)

        lax.fori_loop(0, _NG, _sgroup, 0)

        pltpu.async_copy(xj_v, acc_sh.at[dstc_v], semS, add=True)

    bufA = (srcA_v, dstA_v, eaA_v)
    bufB = (srcB_v, dstB_v, eaB_v)

    _prefetch(0, *bufA)
    _half(0, bufA, bufB, have_prev=False, do_prefetch=True)

    def _pair(h, carry):
        c0 = 1 + 2 * h
        _half(c0, bufB, bufA, have_prev=True, do_prefetch=True)
        _half(c0 + 1, bufA, bufB, have_prev=True, do_prefetch=True)
        return carry

    lax.fori_loop(0, (_NCH - 3) // 2, _pair, 0)
    _half(_NCH - 2, bufB, bufA, have_prev=True, do_prefetch=True)
    _half(_NCH - 1, bufA, bufB, have_prev=True, do_prefetch=False)
    pltpu.make_async_copy(xj_v, acc_sh.at[dstA_v], semS).wait()

    plsc.subcore_barrier()

    pltpu.sync_copy(acc_sh.at[pl.ds(sid * _RPS, _RPS)],
                    out_hbm.at[cid, pl.ds(sid * _RPS, _RPS)])
    pltpu.sync_copy(den_v, den_hbm.at[pl.ds(wid * _N, _N)])


def _gat_sc(xl, xr, idx2, eaflat, wt, att):
    mesh = plsc.VectorSubcoreMesh(core_axis_name="c", subcore_axis_name="s")
    return pl.kernel(
        _gat_sc_body,
        out_type=[
            jax.ShapeDtypeStruct((_NC, _NPAD, _D), jnp.float32),
            jax.ShapeDtypeStruct((_NW * _N,), jnp.float32),
        ],
        mesh=mesh,
        compiler_params=pltpu.CompilerParams(needs_layout_passes=False),
        scratch_types=[
            pltpu.VMEM((_CPT,), jnp.int32),
            pltpu.VMEM((_CPT,), jnp.int32),
            pltpu.VMEM((_DE * _CPT,), jnp.float32),
            pltpu.VMEM((_CPT,), jnp.int32),
            pltpu.VMEM((_CPT,), jnp.int32),
            pltpu.VMEM((_DE * _CPT,), jnp.float32),
            pltpu.VMEM((_CPT, _D), jnp.float32),
            pltpu.VMEM((_CPT, _D), jnp.float32),
            pltpu.VMEM((_DE, _D), jnp.float32),
            pltpu.VMEM((_D,), jnp.float32),
            pltpu.VMEM((_ZR, _D), jnp.float32),
            pltpu.VMEM((128 * 81,), jnp.float32),
            pltpu.VMEM((16 * 17,), jnp.float32),
            pltpu.VMEM((_CPT,), jnp.float32),
            pltpu.VMEM((_N,), jnp.float32),
            pltpu.VMEM_SHARED((_NPAD, _D), jnp.float32),
            pltpu.SemaphoreType.DMA,
            pltpu.SemaphoreType.DMA,
            pltpu.SemaphoreType.DMA,
            pltpu.SemaphoreType.DMA,
        ],
    )(xl, xr, idx2, eaflat, wt, att)


def kernel(x, edge_attr, enc_W, enc_b, enc_ln_g, enc_ln_b, gat_Wl, gat_bl,
           gat_Wr, gat_br, gat_We, gat_att, gat_bias, gln_g, gln_b, dec_W,
           dec_b, edge_index):
    src = edge_index[0].astype(jnp.int32)
    dst = edge_index[1].astype(jnp.int32)
    idx2 = jnp.stack([src.reshape(-1, _CPT), dst.reshape(-1, _CPT)],
                     axis=1).reshape(-1)
    eaflat = jnp.transpose(edge_attr.T.reshape(_DE, -1, _CPT),
                           (1, 0, 2)).reshape(-1)

    y = _enc_call(x, enc_W, enc_b,
                  enc_ln_g.reshape(1, _D), enc_ln_b.reshape(1, _D))
    for i in range(_NB):
        xl, xr = _prep_call(y, gat_Wl[i], gat_bl[i].reshape(1, _D),
                            gat_Wr[i], gat_br[i].reshape(1, _D))
        part, den_flat = _gat_sc(xl, xr, idx2, eaflat,
                                 gat_We[i].T, gat_att[i])
        den_t = den_flat.reshape(_NW, _N).T
        y = _post_call(y, part[0], part[1], den_t,
                       gat_bias[i].reshape(1, _D),
                       gln_g[i].reshape(1, _D), gln_b[i].reshape(1, _D))
    return _dec_call(y, dec_W, dec_b)
